# baseline (speedup 1.0000x reference)
import functools

import jax
import jax.numpy as jnp
from jax import lax
from jax.experimental import pallas as pl
from jax.experimental.pallas import tpu as pltpu
from jax.experimental.pallas import tpu_sc as plsc

NC = 2
NS = 16
NW = NC * NS
LANES = 16
CH = 128
GC = 8


def _sc_mesh():
    return plsc.VectorSubcoreMesh(
        core_axis_name="c", subcore_axis_name="s",
        num_cores=NC, num_subcores=NS)


def _deg_kernel(n_pad, cpw, w=128):
    slc = n_pad // NS

    def body(dst3, ones_hbm, zdeg_hbm, degp, idxv, onesv, degsh):
        c = lax.axis_index("c")
        s = lax.axis_index("s")
        wid = s * NC + c
        pltpu.sync_copy(zdeg_hbm, degsh.at[pl.ds(s * slc, slc)])
        pltpu.sync_copy(ones_hbm, onesv)
        pltpu.sync_copy(dst3.at[wid], idxv)
        plsc.subcore_barrier()

        def chunk(j, carry):
            pltpu.sync_copy(onesv, degsh.at[idxv.at[j]], add=True)
            return carry

        lax.fori_loop(0, cpw, chunk, 0)
        plsc.subcore_barrier()
        pltpu.sync_copy(degsh.at[pl.ds(s * slc, slc)],
                        degp.at[c, pl.ds(s * slc, slc)])

    return pl.kernel(
        body,
        out_type=jax.ShapeDtypeStruct((NC, n_pad, w), jnp.float32),
        mesh=_sc_mesh(),
        scratch_types=[
            pltpu.VMEM((cpw, CH), jnp.int32),
            pltpu.VMEM((CH, w), jnp.float32),
            pltpu.VMEM_SHARED((n_pad, w), jnp.float32),
        ],
    )


def _agg_kernel(n_pad, cpw, d):
    assert cpw % GC == 0 and GC % 2 == 0
    ngrp = cpw // GC
    slc = n_pad // NS

    def body(src3, dst3, g_hbm, zagg_hbm, aggp, srcv, dstv,
             rows0, rows1, aggsh, sem0, sem1):
        c = lax.axis_index("c")
        s = lax.axis_index("s")
        wid = s * NC + c
        pltpu.sync_copy(zagg_hbm, aggsh.at[pl.ds(s * slc, slc)])
        pltpu.sync_copy(src3.at[wid], srcv)
        pltpu.sync_copy(dst3.at[wid, pl.ds(0, GC)], dstv)
        plsc.subcore_barrier()

        pltpu.async_copy(g_hbm.at[srcv.at[0]], rows0, sem0)
        pltpu.async_copy(g_hbm.at[srcv.at[1]], rows1, sem1)

        def group(grp, carry):
            def pair(t, _):
                j0 = grp * GC + 2 * t
                n0 = jnp.minimum(j0 + 2, cpw - 1)
                n1 = jnp.minimum(j0 + 3, cpw - 1)
                pltpu.make_async_copy(g_hbm.at[pl.ds(0, CH)], rows0,
                                      sem0).wait()
                pltpu.sync_copy(rows0, aggsh.at[dstv.at[2 * t]], add=True)
                pltpu.async_copy(g_hbm.at[srcv.at[n0]], rows0, sem0)
                pltpu.make_async_copy(g_hbm.at[pl.ds(0, CH)], rows1,
                                      sem1).wait()
                pltpu.sync_copy(rows1, aggsh.at[dstv.at[2 * t + 1]],
                                add=True)
                pltpu.async_copy(g_hbm.at[srcv.at[n1]], rows1, sem1)
                return _

            lax.fori_loop(0, GC // 2, pair, 0)
            nxt = jnp.minimum(grp + 1, ngrp - 1)
            pltpu.sync_copy(dst3.at[wid, pl.ds(nxt * GC, GC)], dstv)
            return carry

        lax.fori_loop(0, ngrp, group, 0)
        pltpu.make_async_copy(g_hbm.at[pl.ds(0, CH)], rows0, sem0).wait()
        pltpu.make_async_copy(g_hbm.at[pl.ds(0, CH)], rows1, sem1).wait()
        plsc.subcore_barrier()
        pltpu.sync_copy(aggsh.at[pl.ds(s * slc, slc)],
                        aggp.at[c, pl.ds(s * slc, slc)])

    return pl.kernel(
        body,
        out_type=jax.ShapeDtypeStruct((NC, n_pad, d), jnp.float32),
        mesh=_sc_mesh(),
        scratch_types=[
            pltpu.VMEM((cpw, CH), jnp.int32),
            pltpu.VMEM((GC, CH), jnp.int32),
            pltpu.VMEM((CH, d), jnp.float32),
            pltpu.VMEM((CH, d), jnp.float32),
            pltpu.VMEM_SHARED((n_pad, d), jnp.float32),
            pltpu.SemaphoreType.DMA,
            pltpu.SemaphoreType.DMA,
        ],
    )


def _dis(degp_blk):
    deg = degp_blk[0, :, 0:1] + degp_blk[1, :, 0:1]
    return jnp.where(deg > 0, lax.rsqrt(deg), 0.0)


def _mm_scale_body(degp_ref, x_ref, w_ref, g_ref):
    dis = _dis(degp_ref[...])
    h = jnp.dot(x_ref[...], w_ref[...], preferred_element_type=jnp.float32)
    g_ref[...] = dis * h


def _layer2_body(degp_ref, aggp_ref, b_ref, w_ref, g2_ref):
    dis = _dis(degp_ref[...])
    h1 = jnp.maximum(dis * (aggp_ref[0] + aggp_ref[1]) + b_ref[...], 0.0)
    g2_ref[...] = dis * jnp.dot(h1, w_ref[...],
                                preferred_element_type=jnp.float32)


def _final_body(inv_n, degp_ref, aggp_ref, b_ref, out_ref):
    i = pl.program_id(0)
    dis = _dis(degp_ref[...])
    h2 = jnp.maximum(dis * (aggp_ref[0] + aggp_ref[1]) + b_ref[...], 0.0)
    part = jnp.sum(h2, axis=0, keepdims=True) * inv_n

    @pl.when(i == 0)
    def _():
        out_ref[...] = part

    @pl.when(i > 0)
    def _():
        out_ref[...] += part


def _row_block(n, cap):
    best = 8
    for r in range(8, cap + 1, 8):
        if n % r == 0:
            best = r
    return best


def kernel(x, edge_index, W1, b1, W2, b2):
    n, d = x.shape
    e = edge_index.shape[1]
    n_pad = ((n + 1 + 511) // 512) * 512
    e_sl = e + n
    cpw = -(-(-(-e_sl // (NW * CH))) // GC) * GC
    e_pad = NW * cpw * CH

    loop = jnp.arange(n, dtype=jnp.int32)
    pad = jnp.full((e_pad - e_sl,), n, dtype=jnp.int32)
    src3 = jnp.concatenate([edge_index[0], loop, pad]).reshape(NW, cpw, CH)
    dst3 = jnp.concatenate([edge_index[1], loop, pad]).reshape(NW, cpw, CH)
    x_pad = jnp.zeros((n_pad, d), jnp.float32).at[:n].set(x)
    ones_arr = jnp.ones((CH, d), jnp.float32)
    slc = n_pad // NS
    z_deg = jnp.zeros((slc, d), jnp.float32)
    z_agg = jnp.zeros((slc, d), jnp.float32)

    degp = _deg_kernel(n_pad, cpw, d)(dst3, ones_arr, z_deg)

    r2 = 512
    grid2 = n_pad // r2
    degp_spec = pl.BlockSpec((NC, r2, d), lambda i: (0, i, 0))
    aggp_spec = pl.BlockSpec((NC, r2, d), lambda i: (0, i, 0))
    w_spec = pl.BlockSpec((d, d), lambda i: (0, 0))
    b_spec = pl.BlockSpec((1, d), lambda i: (0, 0))
    row_spec = pl.BlockSpec((r2, d), lambda i: (i, 0))

    g1 = pl.pallas_call(
        _mm_scale_body,
        grid=(grid2,),
        in_specs=[degp_spec, row_spec, w_spec],
        out_specs=row_spec,
        out_shape=jax.ShapeDtypeStruct((n_pad, d), jnp.float32),
    )(degp, x_pad, W1)

    agg_fn = _agg_kernel(n_pad, cpw, d)
    aggp1 = agg_fn(src3, dst3, g1, z_agg)

    g2 = pl.pallas_call(
        _layer2_body,
        grid=(grid2,),
        in_specs=[degp_spec, aggp_spec, b_spec, w_spec],
        out_specs=row_spec,
        out_shape=jax.ShapeDtypeStruct((n_pad, d), jnp.float32),
    )(degp, aggp1, b1.reshape(1, d), W2)

    aggp2 = agg_fn(src3, dst3, g2, z_agg)

    r5 = _row_block(n, 2048)
    grid5 = n // r5
    out = pl.pallas_call(
        functools.partial(_final_body, 1.0 / n),
        grid=(grid5,),
        in_specs=[
            pl.BlockSpec((NC, r5, d), lambda i: (0, i, 0)),
            pl.BlockSpec((NC, r5, d), lambda i: (0, i, 0)),
            b_spec,
        ],
        out_specs=pl.BlockSpec((1, d), lambda i: (0, 0)),
        out_shape=jax.ShapeDtypeStruct((1, d), jnp.float32),
    )(degp, aggp2, b2.reshape(1, d))

    return out.reshape(d)


# speedup vs baseline: 1.0425x; 1.0425x over previous
import functools

import jax
import jax.numpy as jnp
from jax import lax
from jax.experimental import pallas as pl
from jax.experimental.pallas import tpu as pltpu
from jax.experimental.pallas import tpu_sc as plsc

NC = 2
NS = 16
NW = NC * NS
LANES = 16
CH = 128
GC = 8


def _sc_mesh():
    return plsc.VectorSubcoreMesh(
        core_axis_name="c", subcore_axis_name="s",
        num_cores=NC, num_subcores=NS)


def _deg_kernel(n_pad, cpw, w=128):
    slc = n_pad // NS

    def body(dst3, ones_hbm, zdeg_hbm, degp, idxv, onesv, degsh):
        c = lax.axis_index("c")
        s = lax.axis_index("s")
        wid = s * NC + c
        pltpu.sync_copy(zdeg_hbm, degsh.at[pl.ds(s * slc, slc)])
        pltpu.sync_copy(ones_hbm, onesv)
        pltpu.sync_copy(dst3.at[wid], idxv)
        plsc.subcore_barrier()

        def chunk(j, carry):
            pltpu.sync_copy(onesv, degsh.at[idxv.at[j]], add=True)
            return carry

        lax.fori_loop(0, cpw, chunk, 0)
        plsc.subcore_barrier()
        pltpu.sync_copy(degsh.at[pl.ds(s * slc, slc)],
                        degp.at[c, pl.ds(s * slc, slc)])

    return pl.kernel(
        body,
        out_type=jax.ShapeDtypeStruct((NC, n_pad, w), jnp.float32),
        mesh=_sc_mesh(),
        scratch_types=[
            pltpu.VMEM((cpw, CH), jnp.int32),
            pltpu.VMEM((CH, w), jnp.float32),
            pltpu.VMEM_SHARED((n_pad, w), jnp.float32),
        ],
    )


def _agg_kernel(n_pad, cpw, d):
    assert cpw % GC == 0 and GC % 2 == 0
    ngrp = cpw // GC
    slc = n_pad // NS

    def body(src3, dst3, g_hbm, zagg_hbm, aggp, srcv, dstv,
             rows0, rows1, aggsh, sem0, sem1):
        c = lax.axis_index("c")
        s = lax.axis_index("s")
        wid = s * NC + c
        pltpu.sync_copy(zagg_hbm, aggsh.at[pl.ds(s * slc, slc)])
        pltpu.sync_copy(src3.at[wid], srcv)
        pltpu.sync_copy(dst3.at[wid, pl.ds(0, GC)], dstv)
        plsc.subcore_barrier()

        def group(grp, carry):
            def pair(t, _):
                j0 = grp * GC + 2 * t
                h0 = pltpu.async_copy(g_hbm.at[srcv.at[j0]], rows0, sem0)
                h1 = pltpu.async_copy(g_hbm.at[srcv.at[j0 + 1]], rows1,
                                      sem1)
                h0.wait()
                h1.wait()
                pltpu.sync_copy(rows0, aggsh.at[dstv.at[2 * t]], add=True)
                pltpu.sync_copy(rows1, aggsh.at[dstv.at[2 * t + 1]],
                                add=True)
                return _

            lax.fori_loop(0, GC // 2, pair, 0)
            nxt = jnp.minimum(grp + 1, ngrp - 1)
            pltpu.sync_copy(dst3.at[wid, pl.ds(nxt * GC, GC)], dstv)
            return carry

        lax.fori_loop(0, ngrp, group, 0)
        plsc.subcore_barrier()
        pltpu.sync_copy(aggsh.at[pl.ds(s * slc, slc)],
                        aggp.at[c, pl.ds(s * slc, slc)])

    return pl.kernel(
        body,
        out_type=jax.ShapeDtypeStruct((NC, n_pad, d), jnp.float32),
        mesh=_sc_mesh(),
        scratch_types=[
            pltpu.VMEM((cpw, CH), jnp.int32),
            pltpu.VMEM((GC, CH), jnp.int32),
            pltpu.VMEM((CH, d), jnp.float32),
            pltpu.VMEM((CH, d), jnp.float32),
            pltpu.VMEM_SHARED((n_pad, d), jnp.float32),
            pltpu.SemaphoreType.DMA,
            pltpu.SemaphoreType.DMA,
        ],
    )


def _dis(degp_blk):
    deg = degp_blk[0, :, 0:1] + degp_blk[1, :, 0:1]
    return jnp.where(deg > 0, lax.rsqrt(deg), 0.0)


def _mm_scale_body(degp_ref, x_ref, w_ref, g_ref):
    dis = _dis(degp_ref[...])
    h = jnp.dot(x_ref[...], w_ref[...], preferred_element_type=jnp.float32)
    g_ref[...] = dis * h


def _layer2_body(degp_ref, aggp_ref, b_ref, w_ref, g2_ref):
    dis = _dis(degp_ref[...])
    h1 = jnp.maximum(dis * (aggp_ref[0] + aggp_ref[1]) + b_ref[...], 0.0)
    g2_ref[...] = dis * jnp.dot(h1, w_ref[...],
                                preferred_element_type=jnp.float32)


def _final_body(inv_n, degp_ref, aggp_ref, b_ref, out_ref):
    i = pl.program_id(0)
    dis = _dis(degp_ref[...])
    h2 = jnp.maximum(dis * (aggp_ref[0] + aggp_ref[1]) + b_ref[...], 0.0)
    part = jnp.sum(h2, axis=0, keepdims=True) * inv_n

    @pl.when(i == 0)
    def _():
        out_ref[...] = part

    @pl.when(i > 0)
    def _():
        out_ref[...] += part


def _row_block(n, cap):
    best = 8
    for r in range(8, cap + 1, 8):
        if n % r == 0:
            best = r
    return best


def kernel(x, edge_index, W1, b1, W2, b2):
    n, d = x.shape
    e = edge_index.shape[1]
    n_pad = ((n + 1 + 511) // 512) * 512
    e_sl = e + n
    cpw = -(-(-(-e_sl // (NW * CH))) // GC) * GC
    e_pad = NW * cpw * CH

    loop = jnp.arange(n, dtype=jnp.int32)
    pad = jnp.full((e_pad - e_sl,), n, dtype=jnp.int32)
    src3 = jnp.concatenate([edge_index[0], loop, pad]).reshape(NW, cpw, CH)
    dst3 = jnp.concatenate([edge_index[1], loop, pad]).reshape(NW, cpw, CH)
    x_pad = jnp.zeros((n_pad, d), jnp.float32).at[:n].set(x)
    ones_arr = jnp.ones((CH, d), jnp.float32)
    slc = n_pad // NS
    z_deg = jnp.zeros((slc, d), jnp.float32)
    z_agg = jnp.zeros((slc, d), jnp.float32)

    degp = _deg_kernel(n_pad, cpw, d)(dst3, ones_arr, z_deg)

    r2 = 512
    grid2 = n_pad // r2
    degp_spec = pl.BlockSpec((NC, r2, d), lambda i: (0, i, 0))
    aggp_spec = pl.BlockSpec((NC, r2, d), lambda i: (0, i, 0))
    w_spec = pl.BlockSpec((d, d), lambda i: (0, 0))
    b_spec = pl.BlockSpec((1, d), lambda i: (0, 0))
    row_spec = pl.BlockSpec((r2, d), lambda i: (i, 0))

    g1 = pl.pallas_call(
        _mm_scale_body,
        grid=(grid2,),
        in_specs=[degp_spec, row_spec, w_spec],
        out_specs=row_spec,
        out_shape=jax.ShapeDtypeStruct((n_pad, d), jnp.float32),
    )(degp, x_pad, W1)

    agg_fn = _agg_kernel(n_pad, cpw, d)
    aggp1 = agg_fn(src3, dst3, g1, z_agg)

    g2 = pl.pallas_call(
        _layer2_body,
        grid=(grid2,),
        in_specs=[degp_spec, aggp_spec, b_spec, w_spec],
        out_specs=row_spec,
        out_shape=jax.ShapeDtypeStruct((n_pad, d), jnp.float32),
    )(degp, aggp1, b1.reshape(1, d), W2)

    aggp2 = agg_fn(src3, dst3, g2, z_agg)

    r5 = _row_block(n, 2048)
    grid5 = n // r5
    out = pl.pallas_call(
        functools.partial(_final_body, 1.0 / n),
        grid=(grid5,),
        in_specs=[
            pl.BlockSpec((NC, r5, d), lambda i: (0, i, 0)),
            pl.BlockSpec((NC, r5, d), lambda i: (0, i, 0)),
            b_spec,
        ],
        out_specs=pl.BlockSpec((1, d), lambda i: (0, 0)),
        out_shape=jax.ShapeDtypeStruct((1, d), jnp.float32),
    )(degp, aggp2, b2.reshape(1, d))

    return out.reshape(d)


# speedup vs baseline: 1.0515x; 1.0086x over previous
import functools

import jax
import jax.numpy as jnp
from jax import lax
from jax.experimental import pallas as pl
from jax.experimental.pallas import tpu as pltpu
from jax.experimental.pallas import tpu_sc as plsc

NC = 2
NS = 16
NW = NC * NS
LANES = 16
CH = 128
GC = 8


def _sc_mesh():
    return plsc.VectorSubcoreMesh(
        core_axis_name="c", subcore_axis_name="s",
        num_cores=NC, num_subcores=NS)


def _deg_kernel(n_pad, cpw, w=128):
    slc = n_pad // NS

    def body(dst3, ones_hbm, zdeg_hbm, degp, idxv, onesv, degsh):
        c = lax.axis_index("c")
        s = lax.axis_index("s")
        wid = s * NC + c
        pltpu.sync_copy(zdeg_hbm, degsh.at[pl.ds(s * slc, slc)])
        pltpu.sync_copy(ones_hbm, onesv)
        pltpu.sync_copy(dst3.at[wid], idxv)
        plsc.subcore_barrier()

        def chunk(j, carry):
            pltpu.sync_copy(onesv, degsh.at[idxv.at[j]], add=True)
            return carry

        lax.fori_loop(0, cpw, chunk, 0)
        plsc.subcore_barrier()
        pltpu.sync_copy(degsh.at[pl.ds(s * slc, slc)],
                        degp.at[c, pl.ds(s * slc, slc)])

    return pl.kernel(
        body,
        out_type=jax.ShapeDtypeStruct((NC, n_pad, w), jnp.float32),
        mesh=_sc_mesh(),
        scratch_types=[
            pltpu.VMEM((cpw, CH), jnp.int32),
            pltpu.VMEM((CH, w), jnp.float32),
            pltpu.VMEM_SHARED((n_pad, w), jnp.float32),
        ],
    )


def _agg_kernel(n_pad, cpw, d):
    assert cpw % GC == 0 and GC % 2 == 0
    ngrp = cpw // GC
    slc = n_pad // NS

    def body(src3, dst3, g_hbm, zagg_hbm, aggp, srcv, dstv,
             rowsv, aggsh, sem):
        c = lax.axis_index("c")
        s = lax.axis_index("s")
        wid = s * NC + c
        pltpu.sync_copy(zagg_hbm, aggsh.at[pl.ds(s * slc, slc)])
        pltpu.sync_copy(src3.at[wid], srcv)
        pltpu.sync_copy(dst3.at[wid], dstv)
        plsc.subcore_barrier()

        def chunk(j, carry):
            pltpu.async_copy(g_hbm.at[srcv.at[j]], rowsv, sem).wait()
            pltpu.sync_copy(rowsv, aggsh.at[dstv.at[j]], add=True)
            return carry

        lax.fori_loop(0, cpw, chunk, 0)
        plsc.subcore_barrier()
        pltpu.sync_copy(aggsh.at[pl.ds(s * slc, slc)],
                        aggp.at[c, pl.ds(s * slc, slc)])

    return pl.kernel(
        body,
        out_type=jax.ShapeDtypeStruct((NC, n_pad, d), jnp.float32),
        mesh=_sc_mesh(),
        scratch_types=[
            pltpu.VMEM((cpw, CH), jnp.int32),
            pltpu.VMEM((cpw, CH), jnp.int32),
            pltpu.VMEM((CH, d), jnp.float32),
            pltpu.VMEM_SHARED((n_pad, d), jnp.float32),
            pltpu.SemaphoreType.DMA,
        ],
    )


def _dis(degp_blk):
    deg = degp_blk[0, :, 0:1] + degp_blk[1, :, 0:1]
    return jnp.where(deg > 0, lax.rsqrt(deg), 0.0)


def _mm_scale_body(degp_ref, x_ref, w_ref, g_ref):
    dis = _dis(degp_ref[...])
    h = jnp.dot(x_ref[...], w_ref[...], preferred_element_type=jnp.float32)
    g_ref[...] = dis * h


def _layer2_body(degp_ref, aggp_ref, b_ref, w_ref, g2_ref):
    dis = _dis(degp_ref[...])
    h1 = jnp.maximum(dis * (aggp_ref[0] + aggp_ref[1]) + b_ref[...], 0.0)
    g2_ref[...] = dis * jnp.dot(h1, w_ref[...],
                                preferred_element_type=jnp.float32)


def _final_body(inv_n, degp_ref, aggp_ref, b_ref, out_ref):
    i = pl.program_id(0)
    dis = _dis(degp_ref[...])
    h2 = jnp.maximum(dis * (aggp_ref[0] + aggp_ref[1]) + b_ref[...], 0.0)
    part = jnp.sum(h2, axis=0, keepdims=True) * inv_n

    @pl.when(i == 0)
    def _():
        out_ref[...] = part

    @pl.when(i > 0)
    def _():
        out_ref[...] += part


def _row_block(n, cap):
    best = 8
    for r in range(8, cap + 1, 8):
        if n % r == 0:
            best = r
    return best


def kernel(x, edge_index, W1, b1, W2, b2):
    n, d = x.shape
    e = edge_index.shape[1]
    n_pad = ((n + 1 + 511) // 512) * 512
    e_sl = e + n
    cpw = -(-(-(-e_sl // (NW * CH))) // GC) * GC
    e_pad = NW * cpw * CH

    loop = jnp.arange(n, dtype=jnp.int32)
    pad = jnp.full((e_pad - e_sl,), n, dtype=jnp.int32)
    src3 = jnp.concatenate([edge_index[0], loop, pad]).reshape(NW, cpw, CH)
    dst3 = jnp.concatenate([edge_index[1], loop, pad]).reshape(NW, cpw, CH)
    x_pad = jnp.zeros((n_pad, d), jnp.float32).at[:n].set(x)
    ones_arr = jnp.ones((CH, d), jnp.float32)
    slc = n_pad // NS
    z_deg = jnp.zeros((slc, d), jnp.float32)
    z_agg = jnp.zeros((slc, d), jnp.float32)

    degp = _deg_kernel(n_pad, cpw, d)(dst3, ones_arr, z_deg)

    r2 = 512
    grid2 = n_pad // r2
    degp_spec = pl.BlockSpec((NC, r2, d), lambda i: (0, i, 0))
    aggp_spec = pl.BlockSpec((NC, r2, d), lambda i: (0, i, 0))
    w_spec = pl.BlockSpec((d, d), lambda i: (0, 0))
    b_spec = pl.BlockSpec((1, d), lambda i: (0, 0))
    row_spec = pl.BlockSpec((r2, d), lambda i: (i, 0))

    g1 = pl.pallas_call(
        _mm_scale_body,
        grid=(grid2,),
        in_specs=[degp_spec, row_spec, w_spec],
        out_specs=row_spec,
        out_shape=jax.ShapeDtypeStruct((n_pad, d), jnp.float32),
    )(degp, x_pad, W1)

    agg_fn = _agg_kernel(n_pad, cpw, d)
    aggp1 = agg_fn(src3, dst3, g1, z_agg)

    g2 = pl.pallas_call(
        _layer2_body,
        grid=(grid2,),
        in_specs=[degp_spec, aggp_spec, b_spec, w_spec],
        out_specs=row_spec,
        out_shape=jax.ShapeDtypeStruct((n_pad, d), jnp.float32),
    )(degp, aggp1, b1.reshape(1, d), W2)

    aggp2 = agg_fn(src3, dst3, g2, z_agg)

    r5 = _row_block(n, 2048)
    grid5 = n // r5
    out = pl.pallas_call(
        functools.partial(_final_body, 1.0 / n),
        grid=(grid5,),
        in_specs=[
            pl.BlockSpec((NC, r5, d), lambda i: (0, i, 0)),
            pl.BlockSpec((NC, r5, d), lambda i: (0, i, 0)),
            b_spec,
        ],
        out_specs=pl.BlockSpec((1, d), lambda i: (0, 0)),
        out_shape=jax.ShapeDtypeStruct((1, d), jnp.float32),
    )(degp, aggp2, b2.reshape(1, d))

    return out.reshape(d)


# speedup vs baseline: 5.2633x; 5.0054x over previous
import functools

import jax
import jax.numpy as jnp
from jax import lax
from jax.experimental import pallas as pl
from jax.experimental.pallas import tpu as pltpu
from jax.experimental.pallas import tpu_sc as plsc

NC = 2
NS = 16
NW = NC * NS
LANES = 16
CH = 128
GC = 8


def _sc_mesh():
    return plsc.VectorSubcoreMesh(
        core_axis_name="c", subcore_axis_name="s",
        num_cores=NC, num_subcores=NS)


def _deg_kernel(n_pad, cpw, w=128):
    slc = n_pad // NS

    def body(dst3, ones_hbm, zdeg_hbm, degp, idxv, onesv, degsh):
        c = lax.axis_index("c")
        s = lax.axis_index("s")
        wid = s * NC + c
        pltpu.sync_copy(zdeg_hbm, degsh.at[pl.ds(s * slc, slc)])
        pltpu.sync_copy(ones_hbm, onesv)
        pltpu.sync_copy(dst3.at[wid], idxv)
        plsc.subcore_barrier()

        def chunk(j, carry):
            pltpu.sync_copy(onesv, degsh.at[idxv.at[j]], add=True)
            return carry

        lax.fori_loop(0, cpw, chunk, 0)
        plsc.subcore_barrier()
        pltpu.sync_copy(degsh.at[pl.ds(s * slc, slc)],
                        degp.at[c, pl.ds(s * slc, slc)])

    return pl.kernel(
        body,
        out_type=jax.ShapeDtypeStruct((NC, n_pad, w), jnp.float32),
        mesh=_sc_mesh(),
        scratch_types=[
            pltpu.VMEM((cpw, CH), jnp.int32),
            pltpu.VMEM((CH, w), jnp.float32),
            pltpu.VMEM_SHARED((n_pad, w), jnp.float32),
        ],
    )


def _agg_kernel(n_pad, cpw, d):
    assert cpw % GC == 0 and GC % 2 == 0
    ngrp = cpw // GC
    slc = n_pad // NS

    def body(src3, dst3, g_hbm, zagg_hbm, aggp, srcv, dstv,
             rowsv, aggsh, sem):
        c = lax.axis_index("c")
        s = lax.axis_index("s")
        wid = s * NC + c
        pltpu.sync_copy(zagg_hbm, aggsh.at[pl.ds(s * slc, slc)])
        pltpu.sync_copy(src3.at[wid], srcv)
        pltpu.sync_copy(dst3.at[wid], dstv)
        plsc.subcore_barrier()

        def chunk(j, carry):
            pltpu.async_copy(g_hbm.at[srcv.at[j]], rowsv, sem).wait()
            pltpu.sync_copy(rowsv, aggsh.at[dstv.at[j]], add=True)
            return carry

        lax.fori_loop(0, cpw, chunk, 0)
        plsc.subcore_barrier()
        pltpu.sync_copy(aggsh.at[pl.ds(s * slc, slc)],
                        aggp.at[c, pl.ds(s * slc, slc)])

    return pl.kernel(
        body,
        out_type=jax.ShapeDtypeStruct((NC, n_pad, d), jnp.float32),
        mesh=_sc_mesh(),
        scratch_types=[
            pltpu.VMEM((cpw, CH), jnp.int32),
            pltpu.VMEM((cpw, CH), jnp.int32),
            pltpu.VMEM((CH, d), jnp.float32),
            pltpu.VMEM_SHARED((n_pad, d), jnp.float32),
            pltpu.SemaphoreType.DMA,
        ],
    )


def _dis(degp_blk):
    deg = degp_blk[0, :, 0:1] + degp_blk[1, :, 0:1]
    return jnp.where(deg > 0, lax.rsqrt(deg), 0.0)


def _mm_scale_body(degp_ref, x_ref, w_ref, g_ref):
    dis = _dis(degp_ref[...])
    h = jnp.dot(x_ref[...], w_ref[...], preferred_element_type=jnp.float32)
    g_ref[...] = dis * h


def _layer2_body(degp_ref, aggp_ref, b_ref, w_ref, g2_ref):
    dis = _dis(degp_ref[...])
    h1 = jnp.maximum(dis * (aggp_ref[0] + aggp_ref[1]) + b_ref[...], 0.0)
    g2_ref[...] = dis * jnp.dot(h1, w_ref[...],
                                preferred_element_type=jnp.float32)


def _final_body(inv_n, degp_ref, aggp_ref, b_ref, out_ref):
    i = pl.program_id(0)
    dis = _dis(degp_ref[...])
    h2 = jnp.maximum(dis * (aggp_ref[0] + aggp_ref[1]) + b_ref[...], 0.0)
    part = jnp.sum(h2, axis=0, keepdims=True) * inv_n

    @pl.when(i == 0)
    def _():
        out_ref[...] = part

    @pl.when(i > 0)
    def _():
        out_ref[...] += part


def _row_block(n, cap):
    best = 8
    for r in range(8, cap + 1, 8):
        if n % r == 0:
            best = r
    return best


def kernel(x, edge_index, W1, b1, W2, b2):
    n, d = x.shape
    e = edge_index.shape[1]
    n_pad = ((n + 1 + 511) // 512) * 512
    e_sl = e + n
    cpw = -(-(-(-e_sl // (NW * CH))) // GC) * GC
    e_pad = NW * cpw * CH

    loop = jnp.arange(n, dtype=jnp.int32)
    pad = n + jnp.arange(e_pad - e_sl, dtype=jnp.int32) % (n_pad - n)
    src3 = jnp.concatenate([edge_index[0], loop, pad]).reshape(NW, cpw, CH)
    dst3 = jnp.concatenate([edge_index[1], loop, pad]).reshape(NW, cpw, CH)
    x_pad = jnp.zeros((n_pad, d), jnp.float32).at[:n].set(x)
    ones_arr = jnp.ones((CH, d), jnp.float32)
    slc = n_pad // NS
    z_deg = jnp.zeros((slc, d), jnp.float32)
    z_agg = jnp.zeros((slc, d), jnp.float32)

    degp = _deg_kernel(n_pad, cpw, d)(dst3, ones_arr, z_deg)

    r2 = 512
    grid2 = n_pad // r2
    degp_spec = pl.BlockSpec((NC, r2, d), lambda i: (0, i, 0))
    aggp_spec = pl.BlockSpec((NC, r2, d), lambda i: (0, i, 0))
    w_spec = pl.BlockSpec((d, d), lambda i: (0, 0))
    b_spec = pl.BlockSpec((1, d), lambda i: (0, 0))
    row_spec = pl.BlockSpec((r2, d), lambda i: (i, 0))

    g1 = pl.pallas_call(
        _mm_scale_body,
        grid=(grid2,),
        in_specs=[degp_spec, row_spec, w_spec],
        out_specs=row_spec,
        out_shape=jax.ShapeDtypeStruct((n_pad, d), jnp.float32),
    )(degp, x_pad, W1)

    agg_fn = _agg_kernel(n_pad, cpw, d)
    aggp1 = agg_fn(src3, dst3, g1, z_agg)

    g2 = pl.pallas_call(
        _layer2_body,
        grid=(grid2,),
        in_specs=[degp_spec, aggp_spec, b_spec, w_spec],
        out_specs=row_spec,
        out_shape=jax.ShapeDtypeStruct((n_pad, d), jnp.float32),
    )(degp, aggp1, b1.reshape(1, d), W2)

    aggp2 = agg_fn(src3, dst3, g2, z_agg)

    r5 = _row_block(n, 2048)
    grid5 = n // r5
    out = pl.pallas_call(
        functools.partial(_final_body, 1.0 / n),
        grid=(grid5,),
        in_specs=[
            pl.BlockSpec((NC, r5, d), lambda i: (0, i, 0)),
            pl.BlockSpec((NC, r5, d), lambda i: (0, i, 0)),
            b_spec,
        ],
        out_specs=pl.BlockSpec((1, d), lambda i: (0, 0)),
        out_shape=jax.ShapeDtypeStruct((1, d), jnp.float32),
    )(degp, aggp2, b2.reshape(1, d))

    return out.reshape(d)


# speedup vs baseline: 7.1618x; 1.3607x over previous
import functools

import jax
import jax.numpy as jnp
from jax import lax
from jax.experimental import pallas as pl
from jax.experimental.pallas import tpu as pltpu
from jax.experimental.pallas import tpu_sc as plsc

NC = 2
NS = 16
NW = NC * NS
LANES = 16
CH = 128
GC = 8


def _sc_mesh():
    return plsc.VectorSubcoreMesh(
        core_axis_name="c", subcore_axis_name="s",
        num_cores=NC, num_subcores=NS)


def _deg_kernel(n_pad, cpw, w=128):
    slc = n_pad // NS

    def body(dst3, ones_hbm, zdeg_hbm, degp, idxv, onesv, degsh):
        c = lax.axis_index("c")
        s = lax.axis_index("s")
        wid = s * NC + c
        pltpu.sync_copy(zdeg_hbm, degsh.at[pl.ds(s * slc, slc)])
        pltpu.sync_copy(ones_hbm, onesv)
        pltpu.sync_copy(dst3.at[wid], idxv)
        plsc.subcore_barrier()

        def chunk(j, carry):
            pltpu.sync_copy(onesv, degsh.at[idxv.at[j]], add=True)
            return carry

        lax.fori_loop(0, cpw, chunk, 0)
        plsc.subcore_barrier()
        pltpu.sync_copy(degsh.at[pl.ds(s * slc, slc)],
                        degp.at[c, pl.ds(s * slc, slc)])

    return pl.kernel(
        body,
        out_type=jax.ShapeDtypeStruct((NC, n_pad, w), jnp.float32),
        mesh=_sc_mesh(),
        scratch_types=[
            pltpu.VMEM((cpw, CH), jnp.int32),
            pltpu.VMEM((CH, w), jnp.float32),
            pltpu.VMEM_SHARED((n_pad, w), jnp.float32),
        ],
    )


def _agg_kernel(n_pad, cpw, d):
    assert cpw % GC == 0 and GC % 2 == 0
    ngrp = cpw // GC
    slc = n_pad // NS

    def body(src3, dst3, g_hbm, zagg_hbm, aggp, srcv, dstv,
             rows0, rows1, aggsh, sem0, sem1):
        c = lax.axis_index("c")
        s = lax.axis_index("s")
        wid = s * NC + c
        pltpu.sync_copy(zagg_hbm, aggsh.at[pl.ds(s * slc, slc)])
        pltpu.sync_copy(src3.at[wid], srcv)
        pltpu.sync_copy(dst3.at[wid, pl.ds(0, GC)], dstv)
        plsc.subcore_barrier()

        pltpu.async_copy(g_hbm.at[srcv.at[0]], rows0, sem0)
        pltpu.async_copy(g_hbm.at[srcv.at[1]], rows1, sem1)

        def group(grp, carry):
            def pair(t, _):
                j0 = grp * GC + 2 * t
                n0 = jnp.minimum(j0 + 2, cpw - 1)
                n1 = jnp.minimum(j0 + 3, cpw - 1)
                pltpu.make_async_copy(g_hbm.at[pl.ds(0, CH)], rows0,
                                      sem0).wait()
                pltpu.sync_copy(rows0, aggsh.at[dstv.at[2 * t]], add=True)
                pltpu.async_copy(g_hbm.at[srcv.at[n0]], rows0, sem0)
                pltpu.make_async_copy(g_hbm.at[pl.ds(0, CH)], rows1,
                                      sem1).wait()
                pltpu.sync_copy(rows1, aggsh.at[dstv.at[2 * t + 1]],
                                add=True)
                pltpu.async_copy(g_hbm.at[srcv.at[n1]], rows1, sem1)
                return _

            lax.fori_loop(0, GC // 2, pair, 0)
            nxt = jnp.minimum(grp + 1, ngrp - 1)
            pltpu.sync_copy(dst3.at[wid, pl.ds(nxt * GC, GC)], dstv)
            return carry

        lax.fori_loop(0, ngrp, group, 0)
        pltpu.make_async_copy(g_hbm.at[pl.ds(0, CH)], rows0, sem0).wait()
        pltpu.make_async_copy(g_hbm.at[pl.ds(0, CH)], rows1, sem1).wait()
        plsc.subcore_barrier()
        pltpu.sync_copy(aggsh.at[pl.ds(s * slc, slc)],
                        aggp.at[c, pl.ds(s * slc, slc)])

    return pl.kernel(
        body,
        out_type=jax.ShapeDtypeStruct((NC, n_pad, d), jnp.float32),
        mesh=_sc_mesh(),
        scratch_types=[
            pltpu.VMEM((cpw, CH), jnp.int32),
            pltpu.VMEM((GC, CH), jnp.int32),
            pltpu.VMEM((CH, d), jnp.float32),
            pltpu.VMEM((CH, d), jnp.float32),
            pltpu.VMEM_SHARED((n_pad, d), jnp.float32),
            pltpu.SemaphoreType.DMA,
            pltpu.SemaphoreType.DMA,
        ],
    )


def _dis(degp_blk):
    deg = degp_blk[0, :, 0:1] + degp_blk[1, :, 0:1]
    return jnp.where(deg > 0, lax.rsqrt(deg), 0.0)


def _mm_scale_body(degp_ref, x_ref, w_ref, g_ref):
    dis = _dis(degp_ref[...])
    h = jnp.dot(x_ref[...], w_ref[...], preferred_element_type=jnp.float32)
    g_ref[...] = dis * h


def _layer2_body(degp_ref, aggp_ref, b_ref, w_ref, g2_ref):
    dis = _dis(degp_ref[...])
    h1 = jnp.maximum(dis * (aggp_ref[0] + aggp_ref[1]) + b_ref[...], 0.0)
    g2_ref[...] = dis * jnp.dot(h1, w_ref[...],
                                preferred_element_type=jnp.float32)


def _final_body(inv_n, degp_ref, aggp_ref, b_ref, out_ref):
    i = pl.program_id(0)
    dis = _dis(degp_ref[...])
    h2 = jnp.maximum(dis * (aggp_ref[0] + aggp_ref[1]) + b_ref[...], 0.0)
    part = jnp.sum(h2, axis=0, keepdims=True) * inv_n

    @pl.when(i == 0)
    def _():
        out_ref[...] = part

    @pl.when(i > 0)
    def _():
        out_ref[...] += part


def _row_block(n, cap):
    best = 8
    for r in range(8, cap + 1, 8):
        if n % r == 0:
            best = r
    return best


def kernel(x, edge_index, W1, b1, W2, b2):
    n, d = x.shape
    e = edge_index.shape[1]
    n_pad = ((n + 1 + 511) // 512) * 512
    e_sl = e + n
    cpw = -(-(-(-e_sl // (NW * CH))) // GC) * GC
    e_pad = NW * cpw * CH

    loop = jnp.arange(n, dtype=jnp.int32)
    pad = n + jnp.arange(e_pad - e_sl, dtype=jnp.int32) % (n_pad - n)
    src3 = jnp.concatenate([edge_index[0], loop, pad]).reshape(NW, cpw, CH)
    dst3 = jnp.concatenate([edge_index[1], loop, pad]).reshape(NW, cpw, CH)
    x_pad = jnp.zeros((n_pad, d), jnp.float32).at[:n].set(x)
    ones_arr = jnp.ones((CH, d), jnp.float32)
    slc = n_pad // NS
    z_deg = jnp.zeros((slc, d), jnp.float32)
    z_agg = jnp.zeros((slc, d), jnp.float32)

    degp = _deg_kernel(n_pad, cpw, d)(dst3, ones_arr, z_deg)

    r2 = 512
    grid2 = n_pad // r2
    degp_spec = pl.BlockSpec((NC, r2, d), lambda i: (0, i, 0))
    aggp_spec = pl.BlockSpec((NC, r2, d), lambda i: (0, i, 0))
    w_spec = pl.BlockSpec((d, d), lambda i: (0, 0))
    b_spec = pl.BlockSpec((1, d), lambda i: (0, 0))
    row_spec = pl.BlockSpec((r2, d), lambda i: (i, 0))

    g1 = pl.pallas_call(
        _mm_scale_body,
        grid=(grid2,),
        in_specs=[degp_spec, row_spec, w_spec],
        out_specs=row_spec,
        out_shape=jax.ShapeDtypeStruct((n_pad, d), jnp.float32),
    )(degp, x_pad, W1)

    agg_fn = _agg_kernel(n_pad, cpw, d)
    aggp1 = agg_fn(src3, dst3, g1, z_agg)

    g2 = pl.pallas_call(
        _layer2_body,
        grid=(grid2,),
        in_specs=[degp_spec, aggp_spec, b_spec, w_spec],
        out_specs=row_spec,
        out_shape=jax.ShapeDtypeStruct((n_pad, d), jnp.float32),
    )(degp, aggp1, b1.reshape(1, d), W2)

    aggp2 = agg_fn(src3, dst3, g2, z_agg)

    r5 = _row_block(n, 2048)
    grid5 = n // r5
    out = pl.pallas_call(
        functools.partial(_final_body, 1.0 / n),
        grid=(grid5,),
        in_specs=[
            pl.BlockSpec((NC, r5, d), lambda i: (0, i, 0)),
            pl.BlockSpec((NC, r5, d), lambda i: (0, i, 0)),
            b_spec,
        ],
        out_specs=pl.BlockSpec((1, d), lambda i: (0, 0)),
        out_shape=jax.ShapeDtypeStruct((1, d), jnp.float32),
    )(degp, aggp2, b2.reshape(1, d))

    return out.reshape(d)
